# baseline (speedup 1.0000x reference)
import functools
import jax
import jax.numpy as jnp
from jax import lax
from jax.experimental import pallas as pl
from jax.experimental.pallas import tpu as pltpu
from jax.experimental.pallas import tpu_sc as plsc

_N = 10000
_E = 160000
_D = 256
_DH = 128
_NC = 2
_NT = 16
_NPAD = 10240
_RPT = _NPAD // _NT
_CH = 128
_GS = 8
_NGRP = 10
_NCHUNK = _NGRP * _GS
_EPT = _NCHUNK * _CH
_EPAD = _EPT * _NT
_BR = 640


def _deg_body(dsts, ones_hbm, out, dst_v, ones_v, acc):
    c = lax.axis_index("c")
    s = lax.axis_index("s")
    pltpu.sync_copy(dsts.at[s], dst_v)
    pltpu.sync_copy(ones_hbm.at[pl.ds(0, _CH)], ones_v)
    pltpu.sync_copy(ones_hbm, acc.at[pl.ds(s * _RPT, _RPT)])
    plsc.subcore_barrier()

    half = _NCHUNK // 2

    def chunk(j, carry):
        r = c * half + j
        pltpu.sync_copy(ones_v, acc.at[dst_v.at[r // _GS, r % _GS]],
                        add=True)
        return carry

    lax.fori_loop(0, half, chunk, 0)
    plsc.subcore_barrier()
    pltpu.sync_copy(acc.at[pl.ds(s * _RPT, _RPT)],
                    out.at[c, pl.ds(s * _RPT, _RPT)])


_deg_kernel = functools.partial(
    pl.kernel,
    out_type=jax.ShapeDtypeStruct((_NC, _NPAD, _DH), jnp.float32),
    mesh=plsc.VectorSubcoreMesh(core_axis_name="c", subcore_axis_name="s"),
    scratch_types=[
        pltpu.VMEM((_NGRP, _GS, _CH), jnp.int32),
        pltpu.VMEM((_CH, _DH), jnp.float32),
        pltpu.VMEM_SHARED((_NPAD, _DH), jnp.float32),
    ],
)(_deg_body)


def _agg_body(table, srcs, dsts, out, gsA, gdA, gsB, gdB, bufa, bufb,
              sema, semb, gsemA, gsemB, acc):
    c = lax.axis_index("c")
    s = lax.axis_index("s")
    pltpu.sync_copy(srcs.at[c, s, 0], gsA)
    pltpu.sync_copy(dsts.at[s, 0], gdA)
    pltpu.sync_copy(table.at[pl.ds(c * _NPAD + s * _RPT, _RPT)],
                    acc.at[pl.ds(s * _RPT, _RPT)])
    plsc.subcore_barrier()

    def issue_idx(g, gs_ref, gd_ref, gsem):
        pltpu.async_copy(srcs.at[c, s, g], gs_ref, gsem)
        pltpu.async_copy(dsts.at[s, g], gd_ref, gsem)

    def wait_idx(gs_ref, gd_ref, gsem):
        pltpu.make_async_copy(srcs.at[c, s, 0], gs_ref, gsem).wait()
        pltpu.make_async_copy(dsts.at[s, 0], gd_ref, gsem).wait()

    bufs = (bufa, bufb)
    sems = (sema, semb)

    def gather(idx_row, buf, sem):
        pltpu.async_copy(table.at[idx_row], buf, sem)

    def wait_gather(buf, sem):
        pltpu.make_async_copy(table.at[gsA.at[0]], buf, sem).wait()

    def section(g, cur_s, cur_d, nxt_s, nxt_d, sem_nxt, my_sem, last):
        wait_idx(nxt_s, nxt_d, sem_nxt)
        for k in range(_GS):
            nrow = cur_s.at[k + 1] if k + 1 < _GS else nxt_s.at[0]
            gather(nrow, bufs[(k + 1) % 2], sems[(k + 1) % 2])
            wait_gather(bufs[k % 2], sems[k % 2])
            pltpu.sync_copy(bufs[k % 2], acc.at[cur_d.at[k]], add=True)
        issue_idx(jnp.minimum(g + 2, _NGRP - 1), cur_s, cur_d, my_sem)

    issue_idx(1, gsB, gdB, gsemB)
    gather(gsA.at[0], bufa, sema)

    def body(q, carry):
        g = 2 * q
        section(g, gsA, gdA, gsB, gdB, gsemB, gsemA, False)
        section(g + 1, gsB, gdB, gsA, gdA, gsemA, gsemB, False)
        return carry

    lax.fori_loop(0, _NGRP // 2, body, 0)
    wait_gather(bufa, sema)
    wait_idx(gsB, gdB, gsemB)
    plsc.subcore_barrier()
    pltpu.sync_copy(acc.at[pl.ds(s * _RPT, _RPT)],
                    out.at[c, pl.ds(s * _RPT, _RPT)])


_agg_kernel = functools.partial(
    pl.kernel,
    out_type=jax.ShapeDtypeStruct((_NC, _NPAD, _DH), jnp.float32),
    mesh=plsc.VectorSubcoreMesh(core_axis_name="c", subcore_axis_name="s"),
    scratch_types=[
        pltpu.VMEM((_GS, _CH), jnp.int32),
        pltpu.VMEM((_GS, _CH), jnp.int32),
        pltpu.VMEM((_GS, _CH), jnp.int32),
        pltpu.VMEM((_GS, _CH), jnp.int32),
        pltpu.VMEM((_CH, _DH), jnp.float32),
        pltpu.VMEM((_CH, _DH), jnp.float32),
        pltpu.SemaphoreType.DMA,
        pltpu.SemaphoreType.DMA,
        pltpu.SemaphoreType.DMA,
        pltpu.SemaphoreType.DMA,
        pltpu.VMEM_SHARED((_NPAD, _DH), jnp.float32),
    ],
)(_agg_body)


def _dinv_of(dg_blk):
    deg = dg_blk[0, :, 0:1] + dg_blk[1, :, 0:1] - 1.0
    return lax.rsqrt(deg)


def _mm1_body(x_ref, w_ref, dg_ref, out_ref):
    dinv = _dinv_of(dg_ref[...])
    h = jnp.dot(x_ref[...], w_ref[...],
                preferred_element_type=jnp.float32)
    ht = h * dinv
    out_ref[0, :, :] = ht[:, :_DH]
    out_ref[1, :, :] = ht[:, _DH:]


def _mm2_body(agg_ref, dg_ref, b_ref, w_ref, out_ref):
    dinv = _dinv_of(dg_ref[...])
    full = agg_ref[...]
    pre = full * dinv[None, :, :] + b_ref[...][:, None, :]
    h1 = jnp.maximum(pre, 0.0)
    h1f = jnp.concatenate([h1[0], h1[1]], axis=1)
    h2 = jnp.dot(h1f, w_ref[...], preferred_element_type=jnp.float32)
    ht2 = h2 * dinv
    out_ref[0, :, :] = ht2[:, :_DH]
    out_ref[1, :, :] = ht2[:, _DH:]


def _head_body(agg_ref, dg_ref, b_ref, wh_ref, bh_ref, out_ref):
    dinv = _dinv_of(dg_ref[...])
    full = agg_ref[...]
    pre = full * dinv[None, :, :] + b_ref[...][:, None, :]
    h2 = jnp.maximum(pre, 0.0)
    h2f = jnp.concatenate([h2[0], h2[1]], axis=1)
    z = jnp.dot(h2f, wh_ref[...], preferred_element_type=jnp.float32)
    out_ref[...] = jax.nn.sigmoid(z + bh_ref[0, 0])


_G = _NPAD // _BR

_split_spec = pl.BlockSpec((2, _BR, _DH), lambda i: (0, i, 0))
_dg_spec = pl.BlockSpec((2, _BR, 16), lambda i: (0, i, 0))
_b_spec = pl.BlockSpec((2, _DH), lambda i: (0, 0))

_mm1 = pl.pallas_call(
    _mm1_body,
    grid=(_G,),
    in_specs=[
        pl.BlockSpec((_BR, _D), lambda i: (i, 0)),
        pl.BlockSpec((_D, _D), lambda i: (0, 0)),
        _dg_spec,
    ],
    out_specs=_split_spec,
    out_shape=jax.ShapeDtypeStruct((2, _NPAD, _DH), jnp.float32),
)

_mm2 = pl.pallas_call(
    _mm2_body,
    grid=(_G,),
    in_specs=[
        _split_spec,
        _dg_spec,
        _b_spec,
        pl.BlockSpec((_D, _D), lambda i: (0, 0)),
    ],
    out_specs=_split_spec,
    out_shape=jax.ShapeDtypeStruct((2, _NPAD, _DH), jnp.float32),
)

_head = pl.pallas_call(
    _head_body,
    grid=(_G,),
    in_specs=[
        _split_spec,
        _dg_spec,
        _b_spec,
        pl.BlockSpec((_D, 1), lambda i: (0, 0)),
        pl.BlockSpec((1, 1), lambda i: (0, 0)),
    ],
    out_specs=pl.BlockSpec((_BR, 1), lambda i: (i, 0)),
    out_shape=jax.ShapeDtypeStruct((_NPAD, 1), jnp.float32),
)


def kernel(x, edge_index, W1, b1, W2, b2, Wh, bh):
    src = edge_index[0]
    dst = edge_index[1]
    pad = _EPAD - _E
    srcp = jnp.concatenate([src, jnp.zeros((pad,), jnp.int32)])
    dstp = jnp.concatenate([dst, jnp.full((pad,), _N, jnp.int32)])
    src3 = srcp.reshape(_NT, _NGRP, _GS, _CH)
    dst3 = dstp.reshape(_NT, _NGRP, _GS, _CH)
    srcs = jnp.stack([src3, src3 + _NPAD])

    ones_rows = jnp.ones((_RPT, _DH), jnp.float32)

    dg = _deg_kernel(dst3, ones_rows)[:, :, :16]

    ht1 = _mm1(x, W1, dg)
    agg1 = _agg_kernel(ht1.reshape(_NC * _NPAD, _DH), srcs, dst3)
    ht2 = _mm2(agg1, dg, b1.reshape(2, _DH), W2)
    agg2 = _agg_kernel(ht2.reshape(_NC * _NPAD, _DH), srcs, dst3)
    risk = _head(agg2, dg, b2.reshape(2, _DH), Wh, bh.reshape(1, 1))
    return risk[:_N, 0]


# speedup vs baseline: 1.0248x; 1.0248x over previous
import functools
import jax
import jax.numpy as jnp
from jax import lax
from jax.experimental import pallas as pl
from jax.experimental.pallas import tpu as pltpu
from jax.experimental.pallas import tpu_sc as plsc

_N = 10000
_E = 160000
_D = 256
_DH = 128
_NC = 2
_NT = 16
_NPAD = 10240
_RPT = _NPAD // _NT
_CH = 128
_GS = 8
_NGRP = 10
_NCHUNK = _NGRP * _GS
_EPT = _NCHUNK * _CH
_EPAD = _EPT * _NT
_BR = 640


def _deg_body(dsts, ones_hbm, out, dst_v, ones_v, acc):
    c = lax.axis_index("c")
    s = lax.axis_index("s")
    pltpu.sync_copy(dsts.at[s], dst_v)
    pltpu.sync_copy(ones_hbm.at[pl.ds(0, _CH)], ones_v)
    pltpu.sync_copy(ones_hbm, acc.at[pl.ds(s * _RPT, _RPT)])
    plsc.subcore_barrier()

    half = _NCHUNK // 2

    def chunk(j, carry):
        r = c * half + j
        pltpu.sync_copy(ones_v, acc.at[dst_v.at[r // _GS, r % _GS]],
                        add=True)
        return carry

    lax.fori_loop(0, half, chunk, 0)
    plsc.subcore_barrier()
    pltpu.sync_copy(acc.at[pl.ds(s * _RPT, _RPT)],
                    out.at[c, pl.ds(s * _RPT, _RPT)])


_deg_kernel = functools.partial(
    pl.kernel,
    out_type=jax.ShapeDtypeStruct((_NC, _NPAD, _DH), jnp.float32),
    mesh=plsc.VectorSubcoreMesh(core_axis_name="c", subcore_axis_name="s"),
    scratch_types=[
        pltpu.VMEM((_NGRP, _GS, _CH), jnp.int32),
        pltpu.VMEM((_CH, _DH), jnp.float32),
        pltpu.VMEM_SHARED((_NPAD, _DH), jnp.float32),
    ],
)(_deg_body)


def _agg_body(table, srcs, dsts, out, gsA, gdA, gsB, gdB, bufa, bufb,
              sema, semb, gsemA, gsemB, acc):
    c = lax.axis_index("c")
    s = lax.axis_index("s")
    pltpu.sync_copy(srcs.at[c, s, 0], gsA)
    pltpu.sync_copy(dsts.at[s, 0], gdA)
    pltpu.sync_copy(table.at[pl.ds(c * _NPAD + s * _RPT, _RPT)],
                    acc.at[pl.ds(s * _RPT, _RPT)])
    plsc.subcore_barrier()

    def issue_idx(g, gs_ref, gd_ref, gsem):
        pltpu.async_copy(srcs.at[c, s, g], gs_ref, gsem)
        pltpu.async_copy(dsts.at[s, g], gd_ref, gsem)

    def wait_idx(gs_ref, gd_ref, gsem):
        pltpu.make_async_copy(srcs.at[c, s, 0], gs_ref, gsem).wait()
        pltpu.make_async_copy(dsts.at[s, 0], gd_ref, gsem).wait()

    bufs = (bufa, bufb)
    sems = (sema, semb)

    def gather(idx_row, buf, sem):
        pltpu.async_copy(table.at[idx_row], buf, sem)

    def wait_gather(buf, sem):
        pltpu.make_async_copy(table.at[gsA.at[0]], buf, sem).wait()

    def section(g, cur_s, cur_d, nxt_s, nxt_d, sem_nxt, my_sem, last):
        wait_idx(nxt_s, nxt_d, sem_nxt)
        for k in range(_GS):
            nrow = cur_s.at[k + 1] if k + 1 < _GS else nxt_s.at[0]
            gather(nrow, bufs[(k + 1) % 2], sems[(k + 1) % 2])
            wait_gather(bufs[k % 2], sems[k % 2])
            pltpu.sync_copy(bufs[k % 2], acc.at[cur_d.at[k]], add=True)
        issue_idx(jnp.minimum(g + 2, _NGRP - 1), cur_s, cur_d, my_sem)

    issue_idx(1, gsB, gdB, gsemB)
    gather(gsA.at[0], bufa, sema)

    def body(q, carry):
        g = 2 * q
        section(g, gsA, gdA, gsB, gdB, gsemB, gsemA, False)
        section(g + 1, gsB, gdB, gsA, gdA, gsemA, gsemB, False)
        return carry

    lax.fori_loop(0, _NGRP // 2, body, 0)
    wait_gather(bufa, sema)
    wait_idx(gsB, gdB, gsemB)
    plsc.subcore_barrier()
    pltpu.sync_copy(acc.at[pl.ds(s * _RPT, _RPT)],
                    out.at[c, pl.ds(s * _RPT, _RPT)])


_agg_kernel = functools.partial(
    pl.kernel,
    out_type=jax.ShapeDtypeStruct((_NC, _NPAD, _DH), jnp.float32),
    mesh=plsc.VectorSubcoreMesh(core_axis_name="c", subcore_axis_name="s"),
    scratch_types=[
        pltpu.VMEM((_GS, _CH), jnp.int32),
        pltpu.VMEM((_GS, _CH), jnp.int32),
        pltpu.VMEM((_GS, _CH), jnp.int32),
        pltpu.VMEM((_GS, _CH), jnp.int32),
        pltpu.VMEM((_CH, _DH), jnp.float32),
        pltpu.VMEM((_CH, _DH), jnp.float32),
        pltpu.SemaphoreType.DMA,
        pltpu.SemaphoreType.DMA,
        pltpu.SemaphoreType.DMA,
        pltpu.SemaphoreType.DMA,
        pltpu.VMEM_SHARED((_NPAD, _DH), jnp.float32),
    ],
)(_agg_body)


def _dinv_of(dg_blk):
    deg = dg_blk[0, :, 0:1] + dg_blk[1, :, 0:1] - 1.0
    return lax.rsqrt(deg)


def _mm0_body(x_ref, w_ref, out_ref):
    out_ref[...] = jnp.dot(x_ref[...], w_ref[...],
                           preferred_element_type=jnp.float32)


def _scale1_body(h_ref, dg_ref, out_ref):
    dinv = _dinv_of(dg_ref[...])
    ht = h_ref[...] * dinv
    out_ref[0, :, :] = ht[:, :_DH]
    out_ref[1, :, :] = ht[:, _DH:]


def _mm2_body(agg_ref, dg_ref, b_ref, w_ref, out_ref):
    dinv = _dinv_of(dg_ref[...])
    full = agg_ref[...]
    pre = full * dinv[None, :, :] + b_ref[...][:, None, :]
    h1 = jnp.maximum(pre, 0.0)
    h1f = jnp.concatenate([h1[0], h1[1]], axis=1)
    h2 = jnp.dot(h1f, w_ref[...], preferred_element_type=jnp.float32)
    ht2 = h2 * dinv
    out_ref[0, :, :] = ht2[:, :_DH]
    out_ref[1, :, :] = ht2[:, _DH:]


def _head_body(agg_ref, dg_ref, b_ref, wh_ref, bh_ref, out_ref):
    dinv = _dinv_of(dg_ref[...])
    full = agg_ref[...]
    pre = full * dinv[None, :, :] + b_ref[...][:, None, :]
    h2 = jnp.maximum(pre, 0.0)
    h2f = jnp.concatenate([h2[0], h2[1]], axis=1)
    z = jnp.dot(h2f, wh_ref[...], preferred_element_type=jnp.float32)
    out_ref[...] = jax.nn.sigmoid(z + bh_ref[0, 0])


_G = _NPAD // _BR

_split_spec = pl.BlockSpec((2, _BR, _DH), lambda i: (0, i, 0))
_dg_spec = pl.BlockSpec((2, _BR, 16), lambda i: (0, i, 0))
_b_spec = pl.BlockSpec((2, _DH), lambda i: (0, 0))

_mm0 = pl.pallas_call(
    _mm0_body,
    grid=(_G,),
    in_specs=[
        pl.BlockSpec((_BR, _D), lambda i: (i, 0)),
        pl.BlockSpec((_D, _D), lambda i: (0, 0)),
    ],
    out_specs=pl.BlockSpec((_BR, _D), lambda i: (i, 0)),
    out_shape=jax.ShapeDtypeStruct((_NPAD, _D), jnp.float32),
)

_scale1 = pl.pallas_call(
    _scale1_body,
    grid=(_G,),
    in_specs=[
        pl.BlockSpec((_BR, _D), lambda i: (i, 0)),
        _dg_spec,
    ],
    out_specs=_split_spec,
    out_shape=jax.ShapeDtypeStruct((2, _NPAD, _DH), jnp.float32),
)

_mm2 = pl.pallas_call(
    _mm2_body,
    grid=(_G,),
    in_specs=[
        _split_spec,
        _dg_spec,
        _b_spec,
        pl.BlockSpec((_D, _D), lambda i: (0, 0)),
    ],
    out_specs=_split_spec,
    out_shape=jax.ShapeDtypeStruct((2, _NPAD, _DH), jnp.float32),
)

_head = pl.pallas_call(
    _head_body,
    grid=(_G,),
    in_specs=[
        _split_spec,
        _dg_spec,
        _b_spec,
        pl.BlockSpec((_D, 1), lambda i: (0, 0)),
        pl.BlockSpec((1, 1), lambda i: (0, 0)),
    ],
    out_specs=pl.BlockSpec((_BR, 1), lambda i: (i, 0)),
    out_shape=jax.ShapeDtypeStruct((_NPAD, 1), jnp.float32),
)


def kernel(x, edge_index, W1, b1, W2, b2, Wh, bh):
    src = edge_index[0]
    dst = edge_index[1]
    pad = _EPAD - _E
    srcp = jnp.concatenate([src, jnp.zeros((pad,), jnp.int32)])
    dstp = jnp.concatenate([dst, jnp.full((pad,), _N, jnp.int32)])
    src3 = srcp.reshape(_NT, _NGRP, _GS, _CH)
    dst3 = dstp.reshape(_NT, _NGRP, _GS, _CH)
    srcs = jnp.stack([src3, src3 + _NPAD])

    ones_rows = jnp.ones((_RPT, _DH), jnp.float32)

    dg = _deg_kernel(dst3, ones_rows)[:, :, :16]

    h1 = _mm0(x, W1)
    ht1 = _scale1(h1, dg)
    agg1 = _agg_kernel(ht1.reshape(_NC * _NPAD, _DH), srcs, dst3)
    ht2 = _mm2(agg1, dg, b1.reshape(2, _DH), W2)
    agg2 = _agg_kernel(ht2.reshape(_NC * _NPAD, _DH), srcs, dst3)
    risk = _head(agg2, dg, b2.reshape(2, _DH), Wh, bh.reshape(1, 1))
    return risk[:_N, 0]
